# Initial kernel scaffold; baseline (speedup 1.0000x reference)
import jax
import jax.numpy as jnp
from jax.experimental import pallas as pl


def kernel(x, edge_index, batch, W0, b0, gamma0, beta0, W1, b1, gamma1, beta1, W2, b2):
    raise NotImplementedError("write your pallas kernel here")


# speedup vs baseline: 14.2925x; 14.2925x over previous
import functools

import jax
import jax.numpy as jnp
from jax import lax
from jax.experimental import pallas as pl
from jax.experimental.pallas import tpu as pltpu
from jax.experimental.pallas import tpu_sc as plsc

N = 10000
E = 320000
G = 100
P = 100
D = 128
D_OUT = 100

NC = 2
NS = 16
NW = NC * NS
EPW = E // NW
CHUNK = 80
NCHUNK = EPW // CHUNK
ROWS_PT = N // NS
ZROWS = 125
IDXB = 2000
BN_C = 1.0 / (1.0 + 1e-5) ** 0.5

def _mesh():
    return plsc.VectorSubcoreMesh(core_axis_name="c", subcore_axis_name="s",
                                  num_cores=NC, num_subcores=NS)


@functools.cache
def _build_sc_pre():
    return functools.partial(
        pl.kernel,
        out_type=(
            jax.ShapeDtypeStruct((NW, 1, N), jnp.float32),
            jax.ShapeDtypeStruct((NW, G, D), jnp.float32),
            jax.ShapeDtypeStruct((NW, G, 16), jnp.float32),
        ),
        mesh=_mesh(),
        compiler_params=pltpu.CompilerParams(needs_layout_passes=False),
        scratch_types=[
            pltpu.VMEM((N,), jnp.float32),
            pltpu.VMEM((G, D), jnp.float32),
            pltpu.VMEM((G, 16), jnp.float32),
            pltpu.VMEM((IDXB,), jnp.int32),
            pltpu.VMEM((IDXB,), jnp.int32),
        ],
    )(_sc_pre_body)


def _sc_pre_body(src_hbm, dst_hbm, deg_out, flag_out, minn_out,
                 deg_v, flag_v, minn_v, src_b, dst_b):
    c = lax.axis_index("c")
    s = lax.axis_index("s")
    wid = s * NC + c
    base = wid * EPW

    zf = jnp.zeros((16,), jnp.float32)

    def zero_deg(i, _):
        deg_v[pl.ds(i * 16, 16)] = zf
        return ()
    lax.fori_loop(0, N // 16, zero_deg, ())

    def zero_flag(i, _):
        flag_v[i // 8, pl.ds((i % 8) * 16, 16)] = zf
        return ()
    lax.fori_loop(0, G * (D // 16), zero_flag, ())

    def init_minn(i, _):
        minn_v[i, :] = jnp.full((16,), float(N), jnp.float32)
        return ()
    lax.fori_loop(0, G, init_minn, ())

    lane = lax.iota(jnp.int32, 16)
    ones = jnp.ones((16,), jnp.float32)

    def outer(ch, _):
        off = base + ch * IDXB
        pltpu.sync_copy(src_hbm.at[pl.ds(off, IDXB)], src_b)
        pltpu.sync_copy(dst_hbm.at[pl.ds(off, IDXB)], dst_b)

        def inner(j, _):
            src16 = src_b[pl.ds(j * 16, 16)]
            dst16 = dst_b[pl.ds(j * 16, 16)]
            plsc.addupdate_scatter(deg_v, [dst16], ones)
            g_src = src16 // P
            p_src = src16 % P
            g_dst = dst16 // P
            p_dst = dst16 % P
            plsc.store_scatter(flag_v, [g_src, p_src], ones)
            same = g_src == g_dst
            plsc.store_scatter(flag_v, [g_dst, p_dst], ones, mask=same)
            cur = plsc.load_gather(minn_v, [g_src, lane])
            cand = jnp.minimum(src16, dst16).astype(jnp.float32)
            plsc.store_scatter(minn_v, [g_src, lane], jnp.minimum(cur, cand))
            return ()
        lax.fori_loop(0, IDXB // 16, inner, ())
        return ()
    lax.fori_loop(0, EPW // IDXB, outer, ())

    pltpu.sync_copy(deg_v, deg_out.at[wid, 0])
    pltpu.sync_copy(flag_v, flag_out.at[wid])
    pltpu.sync_copy(minn_v, minn_out.at[wid])


@functools.cache
def _build_sc_prop():
    return functools.partial(
        pl.kernel,
        out_type=jax.ShapeDtypeStruct((NC, NS, ROWS_PT, D), jnp.float32),
        mesh=_mesh(),
        compiler_params=pltpu.CompilerParams(needs_layout_passes=False),
        scratch_types=[
            pltpu.VMEM_SHARED((N, D), jnp.float32),
            pltpu.VMEM((ZROWS, D), jnp.float32),
            pltpu.VMEM((CHUNK,), jnp.int32),
            pltpu.VMEM((CHUNK,), jnp.int32),
            pltpu.VMEM((CHUNK, D), jnp.float32),
            pltpu.SemaphoreType.DMA,
        ],
    )(_sc_prop_body)


def _sc_prop_body(ht_hbm, src_hbm, dst_hbm, s_out,
                  acc, zero_v, src_v, dst_v, rows_v, sem):
    c = lax.axis_index("c")
    s = lax.axis_index("s")

    zf = jnp.zeros((16,), jnp.float32)

    def zero_buf(i, _):
        zero_v[i // 8, pl.ds((i % 8) * 16, 16)] = zf
        return ()
    lax.fori_loop(0, ZROWS * (D // 16), zero_buf, ())

    def zero_acc(i, _):
        pltpu.sync_copy(zero_v, acc.at[pl.ds(s * ROWS_PT + i * ZROWS, ZROWS)])
        return ()
    lax.fori_loop(0, ROWS_PT // ZROWS, zero_acc, ())

    plsc.subcore_barrier()

    base = c * (E // NC) + s * EPW

    def body(i, _):
        off = base + i * CHUNK
        pltpu.sync_copy(src_hbm.at[pl.ds(off, CHUNK)], src_v)
        pltpu.sync_copy(dst_hbm.at[pl.ds(off, CHUNK)], dst_v)
        pltpu.async_copy(ht_hbm.at[src_v], rows_v, sem).wait()
        pltpu.sync_copy(rows_v, acc.at[dst_v], add=True)
        return ()
    lax.fori_loop(0, NCHUNK, body, ())

    plsc.subcore_barrier()

    pltpu.sync_copy(acc.at[pl.ds(s * ROWS_PT, ROWS_PT)], s_out.at[c, s])


RB = 400
NBLK = N // RB


def _tc0_body(x_ref, w_ref, degp_ref, ht_ref, dinv_ref):
    deg = jnp.sum(degp_ref[...][:, 0, 0, :], axis=0) + 1.0
    dinv = lax.rsqrt(deg)
    y = jnp.dot(x_ref[...], w_ref[...], preferred_element_type=jnp.float32)
    ht_ref[...] = y * dinv[:, None]
    dinv_ref[...] = dinv[:, None]


def _tc0(x, w0, deg_part):
    return pl.pallas_call(
        _tc0_body,
        grid=(NBLK,),
        in_specs=[
            pl.BlockSpec((RB, D), lambda i: (i, 0)),
            pl.BlockSpec((D, D), lambda i: (0, 0)),
            pl.BlockSpec((NW, 1, 1, RB), lambda i: (0, i, 0, 0)),
        ],
        out_specs=[
            pl.BlockSpec((RB, D), lambda i: (i, 0)),
            pl.BlockSpec((RB, 1), lambda i: (i, 0)),
        ],
        out_shape=[
            jax.ShapeDtypeStruct((N, D), jnp.float32),
            jax.ShapeDtypeStruct((N, 1), jnp.float32),
        ],
    )(x, w0, deg_part.reshape(NW, NBLK, 1, RB))


def _tc_mid_body(s_ref, ht_ref, dinv_ref, b_ref, g_ref, be_ref, w_ref, o_ref):
    dinv = dinv_ref[...]
    sm = s_ref[...]
    z = dinv * (sm[0] + sm[1] + ht_ref[...]) + b_ref[...]
    a = jax.nn.relu(z * BN_C * g_ref[...] + be_ref[...])
    y = jnp.dot(a, w_ref[...], preferred_element_type=jnp.float32)
    o_ref[...] = y * dinv


def _tc_mid(s_part, ht, dinv, b, gamma, beta, w_next):
    return pl.pallas_call(
        _tc_mid_body,
        grid=(NBLK,),
        in_specs=[
            pl.BlockSpec((NC, RB, D), lambda i: (0, i, 0)),
            pl.BlockSpec((RB, D), lambda i: (i, 0)),
            pl.BlockSpec((RB, 1), lambda i: (i, 0)),
            pl.BlockSpec((1, D), lambda i: (0, 0)),
            pl.BlockSpec((1, D), lambda i: (0, 0)),
            pl.BlockSpec((1, D), lambda i: (0, 0)),
            pl.BlockSpec((D, D), lambda i: (0, 0)),
        ],
        out_specs=pl.BlockSpec((RB, D), lambda i: (i, 0)),
        out_shape=jax.ShapeDtypeStruct((N, D), jnp.float32),
    )(s_part, ht, dinv, b.reshape(1, D), gamma.reshape(1, D),
      beta.reshape(1, D), w_next)


GB = RB // P


def _tc_fin_body(s_ref, ht_ref, dinv_ref, b_ref, flag_ref, minn_ref, o_ref):
    i = pl.program_id(0)
    sm = s_ref[...]
    h3 = dinv_ref[...] * (sm[0] + sm[1] + ht_ref[...]) + b_ref[...]
    ga = lax.broadcasted_iota(jnp.int32, (GB, RB), 0)
    ra = lax.broadcasted_iota(jnp.int32, (GB, RB), 1) // P
    sel = jnp.where(ga == ra, 1.0 / P, 0.0).astype(jnp.float32)
    pooled = jnp.dot(sel, h3, preferred_element_type=jnp.float32)
    flg = jnp.max(flag_ref[...][:, 0], axis=0)
    mn = jnp.min(jnp.min(minn_ref[...][:, 0], axis=2), axis=0)
    aa = lax.broadcasted_iota(jnp.int32, (GB, D), 0)
    jj = lax.broadcasted_iota(jnp.int32, (GB, D), 1)
    nid = ((i * GB + aa) * P + jj).astype(jnp.float32)
    mask = (flg > 0.5) & (nid != mn[:, None])
    outv = jnp.where(mask, jnp.float32(-1e10), pooled)
    o_ref[...] = outv[None, :, :D_OUT]


def _tc_fin(s_part, ht, dinv, b2p, flag_part, minn_part):
    return pl.pallas_call(
        _tc_fin_body,
        grid=(NBLK,),
        in_specs=[
            pl.BlockSpec((NC, RB, D), lambda i: (0, i, 0)),
            pl.BlockSpec((RB, D), lambda i: (i, 0)),
            pl.BlockSpec((RB, 1), lambda i: (i, 0)),
            pl.BlockSpec((1, D), lambda i: (0, 0)),
            pl.BlockSpec((NW, 1, GB, D), lambda i: (0, i, 0, 0)),
            pl.BlockSpec((NW, 1, GB, 16), lambda i: (0, i, 0, 0)),
        ],
        out_specs=pl.BlockSpec((1, GB, D_OUT), lambda i: (i, 0, 0)),
        out_shape=jax.ShapeDtypeStruct((NBLK, GB, D_OUT), jnp.float32),
    )(s_part, ht, dinv, b2p.reshape(1, D),
      flag_part.reshape(NW, NBLK, GB, D),
      minn_part.reshape(NW, NBLK, GB, 16)).reshape(G, D_OUT)


def kernel(x, edge_index, batch, W0, b0, gamma0, beta0,
           W1, b1, gamma1, beta1, W2, b2):
    src = edge_index[0]
    dst = edge_index[1]

    sc_pre = _build_sc_pre()
    sc_prop = _build_sc_prop()
    deg_part, flag_part, minn_part = sc_pre(src, dst)
    deg_part = deg_part.reshape(NW, N)

    ht0, dinv = _tc0(x, W0, deg_part)
    s0 = sc_prop(ht0, src, dst).reshape(NC, N, D)
    ht1 = _tc_mid(s0, ht0, dinv, b0, gamma0, beta0, W1)
    s1 = sc_prop(ht1, src, dst).reshape(NC, N, D)
    w2p = jnp.pad(W2, ((0, 0), (0, D - D_OUT)))
    ht2 = _tc_mid(s1, ht1, dinv, b1, gamma1, beta1, w2p)
    s2 = sc_prop(ht2, src, dst).reshape(NC, N, D)
    b2p = jnp.pad(b2, (0, D - D_OUT))
    return _tc_fin(s2, ht2, dinv, b2p, flag_part, minn_part)
